# baseline (speedup 1.0000x reference)
import functools

import jax
import jax.numpy as jnp
from jax import lax
from jax.experimental import pallas as pl
from jax.experimental.pallas import tpu as pltpu
from jax.experimental.pallas import tpu_sc as plsc

_D = 64
_B = 4096
_S = 200
_BT = _B * _S
_NC = 2
_NS = 16
_NW = _NC * _NS
_PER_W = _BT // _NW
_IDXW = 128
_C = 640
_RPC = _C // _IDXW
_NCHUNK = _PER_W // _C


def _gather_body(x_hbm, table_hbm, out_hbm, idx_v, rows_v, sem):
    cid = lax.axis_index("c")
    sid = lax.axis_index("s")
    wid = sid * _NC + cid
    base = wid * _PER_W

    def chunk(g, carry):
        off = base + g * _C
        pltpu.sync_copy(x_hbm.at[pl.ds(off, _C)], idx_v)
        copies = [
            pltpu.async_copy(
                table_hbm.at[idx_v.at[pl.ds(r * _IDXW, _IDXW)]],
                rows_v.at[pl.ds(r * _IDXW, _IDXW)],
                sem,
            )
            for r in range(_RPC)
        ]
        for cp in copies:
            cp.wait()
        pltpu.sync_copy(rows_v, out_hbm.at[pl.ds(off, _C)])
        return carry

    lax.fori_loop(0, _NCHUNK, chunk, 0)


@jax.jit
def kernel(x, table):
    xf = x.astype(jnp.int32).reshape(_BT)
    mesh = plsc.VectorSubcoreMesh(core_axis_name="c", subcore_axis_name="s")
    gather = functools.partial(
        pl.kernel,
        mesh=mesh,
        out_type=jax.ShapeDtypeStruct((_BT, _D), jnp.float32),
        scratch_types=[
            pltpu.VMEM((_C,), jnp.int32),
            pltpu.VMEM((_C, _D), jnp.float32),
            pltpu.SemaphoreType.DMA,
        ],
        compiler_params=pltpu.CompilerParams(use_tc_tiling_on_sc=False),
    )(_gather_body)
    out = gather(xf, table)
    return out.reshape(_B, _S, _D)


# speedup vs baseline: 1.0343x; 1.0343x over previous
import functools

import jax
import jax.numpy as jnp
from jax import lax
from jax.experimental import pallas as pl
from jax.experimental.pallas import tpu as pltpu
from jax.experimental.pallas import tpu_sc as plsc

_D = 64
_B = 4096
_S = 200
_NC = 2
_NS = 16
_NW = _NC * _NS
_ROWS_W = _B // _NW
_CR = 4
_SPLITS = ((0, 104), (104, 96))
_NCHUNK = _ROWS_W // _CR
_NBUF = 2


def _gather_body(x_hbm, table_hbm, out_hbm, idx_v, rows_v, gsem, ssem):
    cid = lax.axis_index("c")
    sid = lax.axis_index("s")
    wid = sid * _NC + cid
    row0 = wid * _ROWS_W

    def fire(chunk, b):
        r = row0 + chunk * _CR
        pltpu.sync_copy(x_hbm.at[pl.ds(r, _CR)], idx_v.at[b])
        for rr in range(_CR):
            for s0, w in _SPLITS:
                pltpu.async_copy(
                    table_hbm.at[idx_v.at[b, rr, pl.ds(s0, w)]],
                    rows_v.at[b, rr, pl.ds(s0, w)],
                    gsem.at[b],
                )

    def drain_and_store(chunk, b):
        for rr in range(_CR):
            for s0, w in _SPLITS:
                pltpu.make_async_copy(
                    table_hbm.at[idx_v.at[b, rr, pl.ds(s0, w)]],
                    rows_v.at[b, rr, pl.ds(s0, w)],
                    gsem.at[b],
                ).wait()
        r = row0 + chunk * _CR
        pltpu.async_copy(rows_v.at[b], out_hbm.at[pl.ds(r, _CR)], ssem.at[b])

    def wait_store(b):
        pltpu.make_async_copy(
            rows_v.at[b], out_hbm.at[pl.ds(row0, _CR)], ssem.at[b]
        ).wait()

    for b in range(_NBUF):
        fire(b, b)

    def round_body(i, carry):
        g0 = i * _NBUF
        for b in range(_NBUF):
            drain_and_store(g0 + b, b)
            nxt = g0 + b + _NBUF
            @pl.when(nxt < _NCHUNK)
            def _():
                wait_store(b)
                fire(nxt, b)
        return carry

    lax.fori_loop(0, _NCHUNK // _NBUF, round_body, 0)


@jax.jit
def kernel(x, table):
    xi = x.astype(jnp.int32)
    mesh = plsc.VectorSubcoreMesh(core_axis_name="c", subcore_axis_name="s")
    gather = functools.partial(
        pl.kernel,
        mesh=mesh,
        out_type=jax.ShapeDtypeStruct((_B, _S, _D), jnp.float32),
        scratch_types=[
            pltpu.VMEM((_NBUF, _CR, _S), jnp.int32),
            pltpu.VMEM((_NBUF, _CR, _S, _D), jnp.float32),
            pltpu.SemaphoreType.DMA((_NBUF,)),
            pltpu.SemaphoreType.DMA((_NBUF,)),
        ],
        compiler_params=pltpu.CompilerParams(use_tc_tiling_on_sc=False),
    )(_gather_body)
    return gather(xi, table)
